# baseline (speedup 1.0000x reference)
import jax
import jax.numpy as jnp
from jax import lax
from jax.experimental import pallas as pl

_CODE = 16
_BM = 2048


def _enc_body(x_ref, we_ref, h_ref):
    h_ref[...] = lax.dot_general(x_ref[...], we_ref[...], (((1,), (1,)), ((), ())),
                                 preferred_element_type=jnp.float32)


def kernel(x, W_enc, W_dec):
    B, IN = x.shape
    h = pl.pallas_call(
        _enc_body,
        grid=(B // _BM,),
        in_specs=[
            pl.BlockSpec((_BM, IN), lambda i: (i, 0)),
            pl.BlockSpec((_CODE, IN), lambda i: (0, 0)),
        ],
        out_specs=pl.BlockSpec((_BM, _CODE), lambda i: (i, 0)),
        out_shape=jax.ShapeDtypeStruct((B, _CODE), jnp.float32),
    )(x, W_enc)
    vq = jnp.argmax(h, axis=1)
    return jax.nn.one_hot(vq, _CODE, dtype=jnp.float32) @ W_dec.T
